# baseline (speedup 1.0000x reference)
import jax
import jax.numpy as jnp
from jax.experimental import pallas as pl
from jax.experimental.pallas import tpu as pltpu

_BLOCK_M = 512
_N_CODES = 1024
_DIM = 64


def _vq_body(xt_ref, w_ref, o_ref):
    w = w_ref[...]
    xt = xt_ref[...]
    neg_half_wsq = -0.5 * jnp.sum(w * w, axis=1, keepdims=True)
    laug = jnp.concatenate([w, neg_half_wsq], axis=1)
    ones = jnp.ones((1, xt.shape[1]), jnp.float32)
    raug = jnp.concatenate([xt, ones], axis=0)
    f = jax.lax.dot_general(
        laug, raug, (((1,), (0,)), ((), ())),
        preferred_element_type=jnp.float32)
    m = jnp.max(f, axis=0, keepdims=True)
    ids = jax.lax.broadcasted_iota(jnp.int32, f.shape, 0)
    cand = jnp.where(f == m, ids, _N_CODES)
    idx = jnp.min(cand, axis=0, keepdims=True)
    o_ref[...] = idx[None]


def kernel(x, W):
    n = x.shape[0]
    grid = n // _BLOCK_M
    xt = x.T
    out = pl.pallas_call(
        _vq_body,
        grid=(grid,),
        in_specs=[
            pl.BlockSpec((_DIM, _BLOCK_M), lambda i: (0, i)),
            pl.BlockSpec((_N_CODES, _DIM), lambda i: (0, 0)),
        ],
        out_specs=pl.BlockSpec((1, 1, _BLOCK_M), lambda i: (i, 0, 0)),
        out_shape=jax.ShapeDtypeStruct((grid, 1, _BLOCK_M), jnp.int32),
        compiler_params=pltpu.CompilerParams(
            dimension_semantics=("arbitrary",)),
    )(xt, W)
    return out.reshape(n)


# speedup vs baseline: 1.1817x; 1.1817x over previous
import jax
import jax.numpy as jnp
from jax.experimental import pallas as pl
from jax.experimental.pallas import tpu as pltpu

_BLOCK_M = 512
_N_CODES = 1024
_DIM = 64


def _vq_body(xt_ref, w_ref, o_ref):
    w = w_ref[...]
    xt = xt_ref[...]
    dots = jax.lax.dot_general(
        w, xt, (((1,), (0,)), ((), ())),
        preferred_element_type=jnp.float32)
    f = dots - 0.5 * jnp.sum(w * w, axis=1, keepdims=True)
    vals = f
    ids = jax.lax.broadcasted_iota(jnp.int32, f.shape, 0)
    h = _N_CODES // 2
    while h >= 8:
        a_v, b_v = vals[:h], vals[h:]
        a_i, b_i = ids[:h], ids[h:]
        keep_a = a_v >= b_v
        vals = jnp.where(keep_a, a_v, b_v)
        ids = jnp.where(keep_a, a_i, b_i)
        h //= 2
    m = jnp.max(vals, axis=0, keepdims=True)
    cand = jnp.where(vals == m, ids, _N_CODES)
    idx = jnp.min(cand, axis=0, keepdims=True)
    o_ref[...] = idx[None]


def kernel(x, W):
    n = x.shape[0]
    grid = n // _BLOCK_M
    xt = x.T
    out = pl.pallas_call(
        _vq_body,
        grid=(grid,),
        in_specs=[
            pl.BlockSpec((_DIM, _BLOCK_M), lambda i: (0, i)),
            pl.BlockSpec((_N_CODES, _DIM), lambda i: (0, 0)),
        ],
        out_specs=pl.BlockSpec((1, 1, _BLOCK_M), lambda i: (i, 0, 0)),
        out_shape=jax.ShapeDtypeStruct((grid, 1, _BLOCK_M), jnp.int32),
        compiler_params=pltpu.CompilerParams(
            dimension_semantics=("arbitrary",)),
    )(xt, W)
    return out.reshape(n)


# speedup vs baseline: 1.7282x; 1.4625x over previous
import jax
import jax.numpy as jnp
from jax.experimental import pallas as pl
from jax.experimental.pallas import tpu as pltpu

_BLOCK_M = 1024
_N_CODES = 1024
_DIM = 64


def _vq_body(xt_ref, w_ref, o_ref):
    w = w_ref[...]
    xt = xt_ref[...]
    dots = jax.lax.dot_general(
        w, xt, (((1,), (0,)), ((), ())),
        preferred_element_type=jnp.float32)
    f = dots - 0.5 * jnp.sum(w * w, axis=1, keepdims=True)
    vals = f
    ids = jax.lax.broadcasted_iota(jnp.int32, f.shape, 0)
    h = _N_CODES // 2
    while h >= 8:
        a_v, b_v = vals[:h], vals[h:]
        a_i, b_i = ids[:h], ids[h:]
        keep_a = a_v >= b_v
        vals = jnp.where(keep_a, a_v, b_v)
        ids = jnp.where(keep_a, a_i, b_i)
        h //= 2
    m = jnp.max(vals, axis=0, keepdims=True)
    cand = jnp.where(vals == m, ids, _N_CODES)
    idx = jnp.min(cand, axis=0, keepdims=True)
    o_ref[...] = idx[None]


def kernel(x, W):
    n = x.shape[0]
    grid = n // _BLOCK_M
    xt = x.T
    out = pl.pallas_call(
        _vq_body,
        grid=(grid,),
        in_specs=[
            pl.BlockSpec((_DIM, _BLOCK_M), lambda i: (0, i)),
            pl.BlockSpec((_N_CODES, _DIM), lambda i: (0, 0)),
        ],
        out_specs=pl.BlockSpec((1, 1, _BLOCK_M), lambda i: (i, 0, 0)),
        out_shape=jax.ShapeDtypeStruct((grid, 1, _BLOCK_M), jnp.int32),
        compiler_params=pltpu.CompilerParams(
            dimension_semantics=("arbitrary",)),
    )(xt, W)
    return out.reshape(n)


# speedup vs baseline: 1.9859x; 1.1491x over previous
import jax
import jax.numpy as jnp
from jax.experimental import pallas as pl
from jax.experimental.pallas import tpu as pltpu

_BLOCK_M = 2048
_N_CODES = 1024
_DIM = 64


def _vq_body(xt_ref, w_ref, o_ref):
    w = w_ref[...]
    xt = xt_ref[...]
    dots = jax.lax.dot_general(
        w, xt, (((1,), (0,)), ((), ())),
        preferred_element_type=jnp.float32)
    f = dots - 0.5 * jnp.sum(w * w, axis=1, keepdims=True)
    vals = f
    ids = jax.lax.broadcasted_iota(jnp.int32, f.shape, 0)
    h = _N_CODES // 2
    while h >= 8:
        a_v, b_v = vals[:h], vals[h:]
        a_i, b_i = ids[:h], ids[h:]
        keep_a = a_v >= b_v
        vals = jnp.where(keep_a, a_v, b_v)
        ids = jnp.where(keep_a, a_i, b_i)
        h //= 2
    m = jnp.max(vals, axis=0, keepdims=True)
    cand = jnp.where(vals == m, ids, _N_CODES)
    idx = jnp.min(cand, axis=0, keepdims=True)
    o_ref[...] = idx[None]


def kernel(x, W):
    n = x.shape[0]
    grid = n // _BLOCK_M
    xt = x.T
    out = pl.pallas_call(
        _vq_body,
        grid=(grid,),
        in_specs=[
            pl.BlockSpec((_DIM, _BLOCK_M), lambda i: (0, i)),
            pl.BlockSpec((_N_CODES, _DIM), lambda i: (0, 0)),
        ],
        out_specs=pl.BlockSpec((1, 1, _BLOCK_M), lambda i: (i, 0, 0)),
        out_shape=jax.ShapeDtypeStruct((grid, 1, _BLOCK_M), jnp.int32),
        compiler_params=pltpu.CompilerParams(
            dimension_semantics=("arbitrary",)),
    )(xt, W)
    return out.reshape(n)


# speedup vs baseline: 2.1304x; 1.0728x over previous
import jax
import jax.numpy as jnp
from jax.experimental import pallas as pl
from jax.experimental.pallas import tpu as pltpu

_BLOCK_M = 4096
_N_CODES = 1024
_DIM = 64


def _vq_body(xt_ref, w_ref, o_ref):
    w = w_ref[...]
    xt = xt_ref[...]
    dots = jax.lax.dot_general(
        w, xt, (((1,), (0,)), ((), ())),
        preferred_element_type=jnp.float32)
    f = dots - 0.5 * jnp.sum(w * w, axis=1, keepdims=True)
    vals = f
    ids = jax.lax.broadcasted_iota(jnp.int32, f.shape, 0)
    h = _N_CODES // 2
    while h >= 8:
        a_v, b_v = vals[:h], vals[h:]
        a_i, b_i = ids[:h], ids[h:]
        keep_a = a_v >= b_v
        vals = jnp.where(keep_a, a_v, b_v)
        ids = jnp.where(keep_a, a_i, b_i)
        h //= 2
    m = jnp.max(vals, axis=0, keepdims=True)
    cand = jnp.where(vals == m, ids, _N_CODES)
    idx = jnp.min(cand, axis=0, keepdims=True)
    o_ref[...] = idx[None]


def kernel(x, W):
    n = x.shape[0]
    grid = n // _BLOCK_M
    xt = x.T
    out = pl.pallas_call(
        _vq_body,
        grid=(grid,),
        in_specs=[
            pl.BlockSpec((_DIM, _BLOCK_M), lambda i: (0, i)),
            pl.BlockSpec((_N_CODES, _DIM), lambda i: (0, 0)),
        ],
        out_specs=pl.BlockSpec((1, 1, _BLOCK_M), lambda i: (i, 0, 0)),
        out_shape=jax.ShapeDtypeStruct((grid, 1, _BLOCK_M), jnp.int32),
        compiler_params=pltpu.CompilerParams(
            dimension_semantics=("arbitrary",)),
    )(xt, W)
    return out.reshape(n)


# speedup vs baseline: 2.1517x; 1.0100x over previous
import jax
import jax.numpy as jnp
from jax.experimental import pallas as pl
from jax.experimental.pallas import tpu as pltpu

_BLOCK_M = 8192
_N_CODES = 1024
_DIM = 64


def _vq_body(xt_ref, w_ref, o_ref):
    w = w_ref[...]
    xt = xt_ref[...]
    dots = jax.lax.dot_general(
        w, xt, (((1,), (0,)), ((), ())),
        preferred_element_type=jnp.float32)
    f = dots - 0.5 * jnp.sum(w * w, axis=1, keepdims=True)
    vals = f
    ids = jax.lax.broadcasted_iota(jnp.int32, f.shape, 0)
    h = _N_CODES // 2
    while h >= 8:
        a_v, b_v = vals[:h], vals[h:]
        a_i, b_i = ids[:h], ids[h:]
        keep_a = a_v >= b_v
        vals = jnp.where(keep_a, a_v, b_v)
        ids = jnp.where(keep_a, a_i, b_i)
        h //= 2
    m = jnp.max(vals, axis=0, keepdims=True)
    cand = jnp.where(vals == m, ids, _N_CODES)
    idx = jnp.min(cand, axis=0, keepdims=True)
    o_ref[...] = idx[None]


def kernel(x, W):
    n = x.shape[0]
    grid = n // _BLOCK_M
    xt = x.T
    out = pl.pallas_call(
        _vq_body,
        grid=(grid,),
        in_specs=[
            pl.BlockSpec((_DIM, _BLOCK_M), lambda i: (0, i)),
            pl.BlockSpec((_N_CODES, _DIM), lambda i: (0, 0)),
        ],
        out_specs=pl.BlockSpec((1, 1, _BLOCK_M), lambda i: (i, 0, 0)),
        out_shape=jax.ShapeDtypeStruct((grid, 1, _BLOCK_M), jnp.int32),
        compiler_params=pltpu.CompilerParams(
            dimension_semantics=("arbitrary",)),
    )(xt, W)
    return out.reshape(n)
